# baseline (speedup 1.0000x reference)
import functools

import jax
import jax.numpy as jnp
from jax import lax
from jax.experimental import pallas as pl
from jax.experimental.pallas import tpu as pltpu
from jax.experimental.pallas import tpu_sc as plsc

N = 10000
E = 320000
F = 128
H = 512
G = 64
NC = 2
NS = 16
NW = NC * NS
B = 128
NB = 81
EPW = B * NB
E_PAD = NW * EPW
WT = 10
RPT = N // WT
BN = 2000


D = 3


def _sc_agg(table, idx_pack, w_pack, C):
    mesh = plsc.VectorSubcoreMesh(core_axis_name="c", subcore_axis_name="s")

    @functools.partial(
        pl.kernel,
        out_type=jax.ShapeDtypeStruct((C * NC * N, F), jnp.float32),
        mesh=mesh,
        scratch_types=(
            [pltpu.VMEM((2, B), jnp.int32) for _ in range(D)]
            + [pltpu.VMEM((B,), jnp.float32) for _ in range(D)]
            + [pltpu.VMEM((B, F), jnp.float32) for _ in range(D)]
            + [pltpu.VMEM_SHARED((N, F), jnp.float32)]
            + [pltpu.SemaphoreType.DMA for _ in range(D)]
        ),
    )
    def k(table_h, idx_h, w_h, out_h,
          i0, i1, i2, w0, w1, w2, r0, r1, r2,
          acc_s,
          g0, g1, g2):
        idxs = [i0, i1, i2]
        ws = [w0, w1, w2]
        rows = [r0, r1, r2]
        semg = [g0, g1, g2]
        cid = lax.axis_index("c")
        sid = lax.axis_index("s")
        wid = sid * NC + cid
        kbase = wid * NB

        def fetch(g, b, coff):
            pltpu.sync_copy(idx_h.at[kbase + g], idxs[b])
            pltpu.sync_copy(w_h.at[kbase + g], ws[b])
            for q in range(B // 16):
                sl = pl.ds(q * 16, 16)
                idxs[b][0, sl] = idxs[b][0, sl] + coff
            pltpu.async_copy(table_h.at[idxs[b].at[0]], rows[b], semg[b])

        def chunk_body(c, carry):
            coff = c * N
            @pl.when(sid < WT)
            def _zero():
                def zrow(j, carry2):
                    for f8 in range(F // 16):
                        rows[0][j, pl.ds(f8 * 16, 16)] = (
                            jnp.zeros((16,), jnp.float32))
                    return carry2
                lax.fori_loop(0, B, zrow, 0)
                for z in range(7):
                    pltpu.sync_copy(rows[0],
                                    acc_s.at[pl.ds(sid * RPT + z * B, B)])
                pltpu.sync_copy(rows[0].at[pl.ds(0, RPT - 7 * B)],
                                acc_s.at[pl.ds(sid * RPT + 7 * B,
                                               RPT - 7 * B)])
            plsc.subcore_barrier()

            fetch(0, 0, coff)
            fetch(1, 1, coff)

            def step(t, carry):
                for b in range(D):
                    g = t * D + b
                    bf = (b + 2) % D
                    pltpu.make_async_copy(table_h.at[idxs[b].at[0]],
                                          rows[b], semg[b]).wait()
                    fetch((g + 2) % NB, bf, coff)

                    def scale(q, inner):
                        wvec = ws[b][pl.ds(q * 16, 16)]
                        for j16 in range(16):
                            wj = wvec[j16]
                            j = q * 16 + j16
                            for f8 in range(F // 16):
                                sl = pl.ds(f8 * 16, 16)
                                rows[b][j, sl] = rows[b][j, sl] * wj
                        return inner
                    lax.fori_loop(0, B // 16, scale, 0)

                    pltpu.sync_copy(rows[b], acc_s.at[idxs[b].at[1]],
                                    add=True)
                return carry
            lax.fori_loop(0, NB // D, step, 0)

            for b in (0, 1):
                pltpu.make_async_copy(table_h.at[idxs[b].at[0]],
                                      rows[b], semg[b]).wait()
            plsc.subcore_barrier()

            @pl.when(sid < WT)
            def _writeout():
                out_row = (c * NC + cid) * N + sid * RPT
                pltpu.sync_copy(acc_s.at[pl.ds(sid * RPT, RPT)],
                                out_h.at[pl.ds(out_row, RPT)])
            plsc.subcore_barrier()
            return carry
        lax.fori_loop(0, C, chunk_body, 0)

    return k(table, idx_pack, w_pack)


def _tc_layer1(acc, x, W_rel, W_root, b_rel):
    def body(acc_ref, x_ref, wr_ref, wt_ref, b_ref, out_ref):
        agg = acc_ref[0] + acc_ref[1]
        pre = lax.dot_general(agg, wr_ref[...], (((1,), (1,)), ((), ())),
                              preferred_element_type=jnp.float32, precision=lax.Precision.HIGHEST)
        pre = pre + lax.dot_general(x_ref[...], wt_ref[...],
                                    (((1,), (1,)), ((), ())),
                                    preferred_element_type=jnp.float32, precision=lax.Precision.HIGHEST)
        out_ref[0] = jnp.maximum(pre + b_ref[0], 0.0)

    return pl.pallas_call(
        body,
        grid=(4, N // BN),
        in_specs=[
            pl.BlockSpec((2, BN, F), lambda c, i: (0, i, 0)),
            pl.BlockSpec((BN, F), lambda c, i: (i, 0)),
            pl.BlockSpec((F, F), lambda c, i: (c, 0)),
            pl.BlockSpec((F, F), lambda c, i: (c, 0)),
            pl.BlockSpec((1, 1, F), lambda c, i: (c, 0, 0)),
        ],
        out_specs=pl.BlockSpec((1, BN, F), lambda c, i: (c, i, 0)),
        out_shape=jax.ShapeDtypeStruct((4, N, F), jnp.float32),
    )(acc, x, W_rel, W_root, b_rel.reshape(4, 1, F))


def _tc_layer2_pool(acc2, h1b, W_rel, W_root, b_rel, batch2d):
    def body(acc_ref, h1_ref, wr_ref, wt_ref, b_ref, bat_ref,
             pooled_ref, counts_ref):
        i = pl.program_id(0)
        wr = wr_ref[...]
        wt = wt_ref[...]
        total = jnp.zeros((BN, H), jnp.float32)
        for c in range(4):
            aggc = acc_ref[c, 0] + acc_ref[c, 1]
            total = total + lax.dot_general(
                aggc, wr[:, c * F:(c + 1) * F], (((1,), (1,)), ((), ())),
                preferred_element_type=jnp.float32, precision=lax.Precision.HIGHEST)
            total = total + lax.dot_general(
                h1_ref[c], wt[:, c * F:(c + 1) * F], (((1,), (1,)), ((), ())),
                preferred_element_type=jnp.float32, precision=lax.Precision.HIGHEST)
        h2 = jnp.maximum(total + b_ref[...], 0.0)
        bat = bat_ref[0, 0]
        gids = lax.broadcasted_iota(jnp.int32, (G, BN), 0)
        mask = (bat[None, :] == gids).astype(jnp.float32)
        psum = lax.dot_general(mask, h2, (((1,), (0,)), ((), ())),
                               preferred_element_type=jnp.float32, precision=lax.Precision.HIGHEST)
        cnt = lax.dot_general(mask, jnp.ones((BN, H), jnp.float32),
                              (((1,), (0,)), ((), ())),
                              preferred_element_type=jnp.float32, precision=lax.Precision.HIGHEST)

        @pl.when(i == 0)
        def _():
            pooled_ref[...] = jnp.zeros_like(pooled_ref)
            counts_ref[...] = jnp.zeros_like(counts_ref)

        pooled_ref[...] += psum
        counts_ref[...] += cnt

    return pl.pallas_call(
        body,
        grid=(N // BN,),
        in_specs=[
            pl.BlockSpec((4, 2, BN, F), lambda i: (0, 0, i, 0)),
            pl.BlockSpec((4, BN, F), lambda i: (0, i, 0)),
            pl.BlockSpec((H, H), lambda i: (0, 0)),
            pl.BlockSpec((H, H), lambda i: (0, 0)),
            pl.BlockSpec((1, H), lambda i: (0, 0)),
            pl.BlockSpec((1, 1, BN), lambda i: (i, 0, 0)),
        ],
        out_specs=[
            pl.BlockSpec((G, H), lambda i: (0, 0)),
            pl.BlockSpec((G, H), lambda i: (0, 0)),
        ],
        out_shape=[
            jax.ShapeDtypeStruct((G, H), jnp.float32),
            jax.ShapeDtypeStruct((G, H), jnp.float32),
        ],
    )(acc2, h1b, W_rel, W_root, b_rel.reshape(1, H), batch2d)


def _tc_head(pooled, counts, W1, b1, W2, b2, W3, b3):
    def body(p_ref, c_ref, w1_ref, b1_ref, w2_ref, b2_ref, w3_ref, b3_ref,
             out_ref):
        pm = p_ref[...] / jnp.maximum(c_ref[...], 1.0)
        z = lax.dot_general(pm, w1_ref[...], (((1,), (1,)), ((), ())),
                            preferred_element_type=jnp.float32, precision=lax.Precision.HIGHEST) + b1_ref[...]
        z = jnp.maximum(z, 0.0)
        z = lax.dot_general(z, w2_ref[...], (((1,), (1,)), ((), ())),
                            preferred_element_type=jnp.float32, precision=lax.Precision.HIGHEST) + b2_ref[...]
        z = jnp.maximum(z, 0.0)
        out_ref[...] = (jnp.sum(z * w3_ref[...], axis=1, keepdims=True)
                        + b3_ref[...])

    W2p = jnp.zeros((128, G), jnp.float32).at[:16].set(W2)
    b2p = jnp.zeros((1, 128), jnp.float32).at[:, :16].set(b2.reshape(1, 16))
    W3p = jnp.zeros((1, 128), jnp.float32).at[:, :16].set(W3)
    return pl.pallas_call(
        body,
        out_shape=jax.ShapeDtypeStruct((G, 1), jnp.float32),
    )(pooled, counts, W1, b1.reshape(1, G), W2p, b2p, W3p, b3.reshape(1, 1))


def kernel(x, edge_index, edge_attr, batch,
           W_rel1, b_rel1, W_root1,
           W_rel2, b_rel2, W_root2,
           W1, b1, W2, b2, W3, b3):
    pad = E_PAD - E
    src_p = jnp.concatenate([edge_index[0], jnp.zeros((pad,), jnp.int32)])
    dst_p = jnp.concatenate([edge_index[1], jnp.zeros((pad,), jnp.int32)])
    w_p = jnp.concatenate([edge_attr, jnp.zeros((pad,), jnp.float32)])
    idx_pack = jnp.stack([src_p.reshape(NW * NB, B),
                          dst_p.reshape(NW * NB, B)], axis=1)
    w_pack = w_p.reshape(NW * NB, B)

    acc1 = _sc_agg(x, idx_pack, w_pack, C=1).reshape(2, N, F)
    h1b = _tc_layer1(acc1, x, W_rel1, W_root1, b_rel1)
    acc2 = _sc_agg(h1b.reshape(4 * N, F), idx_pack, w_pack,
                   C=4).reshape(4, 2, N, F)
    pooled, counts = _tc_layer2_pool(acc2, h1b, W_rel2, W_root2, b_rel2,
                                     batch.reshape(N // BN, 1, BN))
    return _tc_head(pooled, counts, W1, b1, W2, b2, W3, b3)


# speedup vs baseline: 1.2287x; 1.2287x over previous
import functools

import jax
import jax.numpy as jnp
from jax import lax
from jax.experimental import pallas as pl
from jax.experimental.pallas import tpu as pltpu
from jax.experimental.pallas import tpu_sc as plsc

N = 10000
E = 320000
F = 128
H = 512
G = 64
NC = 2
NS = 16
NW = NC * NS
B = 128
NB = 80
EPW = B * NB
E_PAD = NW * EPW
WT = 10
RPT = N // WT
BN = 2000


D = 2
SUP = 8
NSUP = NB // SUP


def _sc_agg(table, idx_pack, w_pack, C):
    mesh = plsc.VectorSubcoreMesh(core_axis_name="c", subcore_axis_name="s")

    @functools.partial(
        pl.kernel,
        out_type=jax.ShapeDtypeStruct((C * NC * N, F), jnp.float32),
        mesh=mesh,
        scratch_types=(
            [pltpu.VMEM((SUP, 2, B), jnp.int32) for _ in range(2)]
            + [pltpu.VMEM((SUP, B), jnp.float32) for _ in range(2)]
            + [pltpu.VMEM((B, F), jnp.float32) for _ in range(D)]
            + [pltpu.VMEM_SHARED((N, F), jnp.float32)]
            + [pltpu.SemaphoreType.DMA for _ in range(2 * D + 2)]
        ),
    )
    def k(table_h, idx_h, w_h, out_h,
          i0, i1, w0, w1, r0, r1,
          acc_s,
          g0, g1, s0, s1, n0, n1):
        idxs = [i0, i1]
        ws = [w0, w1]
        rows = [r0, r1]
        semg = [g0, g1]
        sems = [s0, s1]
        semi = [n0, n1]
        cid = lax.axis_index("c")
        sid = lax.axis_index("s")
        wid = sid * NC + cid
        kbase = wid * NB

        def super_fetch_start(sb, p):
            off = kbase + sb * SUP
            pltpu.async_copy(idx_h.at[pl.ds(off, SUP)], idxs[p], semi[p])
            pltpu.async_copy(w_h.at[pl.ds(off, SUP)], ws[p], semi[p])

        def super_fetch_finish(sb, p, coff):
            off = kbase + sb * SUP
            pltpu.make_async_copy(idx_h.at[pl.ds(off, SUP)], idxs[p],
                                  semi[p]).wait()
            pltpu.make_async_copy(w_h.at[pl.ds(off, SUP)], ws[p],
                                  semi[p]).wait()
            for b8 in range(SUP):
                for q in range(B // 16):
                    sl = pl.ds(q * 16, 16)
                    idxs[p][b8, 0, sl] = idxs[p][b8, 0, sl] + coff

        def chunk_body(c, carry):
            coff = c * N
            @pl.when(sid < WT)
            def _zero():
                def zrow(j, carry2):
                    for f8 in range(F // 16):
                        rows[0][j, pl.ds(f8 * 16, 16)] = (
                            jnp.zeros((16,), jnp.float32))
                    return carry2
                lax.fori_loop(0, B, zrow, 0)
                for z in range(7):
                    pltpu.sync_copy(rows[0],
                                    acc_s.at[pl.ds(sid * RPT + z * B, B)])
                pltpu.sync_copy(rows[0].at[pl.ds(0, RPT - 7 * B)],
                                acc_s.at[pl.ds(sid * RPT + 7 * B,
                                               RPT - 7 * B)])
            plsc.subcore_barrier()

            super_fetch_start(0, 0)
            super_fetch_finish(0, 0, coff)
            pltpu.async_copy(table_h.at[idxs[0].at[0, 0]], rows[0], semg[0])
            pltpu.async_copy(table_h.at[idxs[0].at[0, 0]], rows[1], sems[1])

            def step(t, carry):
              for p in range(2):
                sb = t * 2 + p
                for b8 in range(SUP):
                    r = b8 % 2
                    rr = 1 - r
                    pltpu.make_async_copy(table_h.at[idxs[0].at[0, 0]],
                                          rows[r], semg[r]).wait()
                    if b8 == 0:
                        pidx = idxs[1 - p].at[SUP - 1, 1]
                    else:
                        pidx = idxs[p].at[b8 - 1, 1]
                    pltpu.make_async_copy(rows[rr], acc_s.at[pidx],
                                          sems[rr]).wait()
                    if b8 == 0:
                        super_fetch_start((sb + 1) % NSUP, 1 - p)
                    if b8 == SUP - 1:
                        super_fetch_finish((sb + 1) % NSUP, 1 - p,
                                           coff)
                        gidx = idxs[1 - p].at[0, 0]
                    else:
                        gidx = idxs[p].at[b8 + 1, 0]
                    pltpu.async_copy(table_h.at[gidx], rows[rr], semg[rr])

                    def scale(q, inner):
                        wvec = ws[p][b8, pl.ds(q * 16, 16)]
                        for j16 in range(16):
                            wj = wvec[j16]
                            j = q * 16 + j16
                            for f8 in range(F // 16):
                                sl = pl.ds(f8 * 16, 16)
                                rows[r][j, sl] = rows[r][j, sl] * wj
                        return inner
                    lax.fori_loop(0, B // 16, scale, 0)

                    pltpu.async_copy(rows[r], acc_s.at[idxs[p].at[b8, 1]],
                                     sems[r], add=True)
              return carry
            lax.fori_loop(0, NSUP // 2, step, 0)

            pltpu.make_async_copy(table_h.at[idxs[0].at[0, 0]],
                                  rows[0], semg[0]).wait()
            pltpu.make_async_copy(rows[1], acc_s.at[idxs[1].at[SUP - 1, 1]],
                                  sems[1]).wait()
            plsc.subcore_barrier()

            @pl.when(sid < WT)
            def _writeout():
                out_row = (c * NC + cid) * N + sid * RPT
                pltpu.sync_copy(acc_s.at[pl.ds(sid * RPT, RPT)],
                                out_h.at[pl.ds(out_row, RPT)])
            plsc.subcore_barrier()
            return carry
        lax.fori_loop(0, C, chunk_body, 0)

    return k(table, idx_pack, w_pack)


def _tc_layer1(acc, x, W_rel, W_root, b_rel):
    def body(acc_ref, x_ref, wr_ref, wt_ref, b_ref, out_ref):
        agg = acc_ref[0] + acc_ref[1]
        pre = lax.dot_general(agg, wr_ref[...], (((1,), (1,)), ((), ())),
                              preferred_element_type=jnp.float32, precision=lax.Precision.HIGHEST)
        pre = pre + lax.dot_general(x_ref[...], wt_ref[...],
                                    (((1,), (1,)), ((), ())),
                                    preferred_element_type=jnp.float32, precision=lax.Precision.HIGHEST)
        out_ref[0] = jnp.maximum(pre + b_ref[0], 0.0)

    return pl.pallas_call(
        body,
        grid=(4, N // BN),
        in_specs=[
            pl.BlockSpec((2, BN, F), lambda c, i: (0, i, 0)),
            pl.BlockSpec((BN, F), lambda c, i: (i, 0)),
            pl.BlockSpec((F, F), lambda c, i: (c, 0)),
            pl.BlockSpec((F, F), lambda c, i: (c, 0)),
            pl.BlockSpec((1, 1, F), lambda c, i: (c, 0, 0)),
        ],
        out_specs=pl.BlockSpec((1, BN, F), lambda c, i: (c, i, 0)),
        out_shape=jax.ShapeDtypeStruct((4, N, F), jnp.float32),
    )(acc, x, W_rel, W_root, b_rel.reshape(4, 1, F))


def _tc_layer2_pool(acc2, h1b, W_rel, W_root, b_rel, batch2d):
    def body(acc_ref, h1_ref, wr_ref, wt_ref, b_ref, bat_ref,
             pooled_ref, counts_ref):
        i = pl.program_id(0)
        wr = wr_ref[...]
        wt = wt_ref[...]
        total = jnp.zeros((BN, H), jnp.float32)
        for c in range(4):
            aggc = acc_ref[c, 0] + acc_ref[c, 1]
            total = total + lax.dot_general(
                aggc, wr[:, c * F:(c + 1) * F], (((1,), (1,)), ((), ())),
                preferred_element_type=jnp.float32, precision=lax.Precision.HIGHEST)
            total = total + lax.dot_general(
                h1_ref[c], wt[:, c * F:(c + 1) * F], (((1,), (1,)), ((), ())),
                preferred_element_type=jnp.float32, precision=lax.Precision.HIGHEST)
        h2 = jnp.maximum(total + b_ref[...], 0.0)
        bat = bat_ref[0, 0]
        gids = lax.broadcasted_iota(jnp.int32, (G, BN), 0)
        mask = (bat[None, :] == gids).astype(jnp.float32)
        psum = lax.dot_general(mask, h2, (((1,), (0,)), ((), ())),
                               preferred_element_type=jnp.float32, precision=lax.Precision.HIGHEST)
        cnt = lax.dot_general(mask, jnp.ones((BN, H), jnp.float32),
                              (((1,), (0,)), ((), ())),
                              preferred_element_type=jnp.float32, precision=lax.Precision.HIGHEST)

        @pl.when(i == 0)
        def _():
            pooled_ref[...] = jnp.zeros_like(pooled_ref)
            counts_ref[...] = jnp.zeros_like(counts_ref)

        pooled_ref[...] += psum
        counts_ref[...] += cnt

    return pl.pallas_call(
        body,
        grid=(N // BN,),
        in_specs=[
            pl.BlockSpec((4, 2, BN, F), lambda i: (0, 0, i, 0)),
            pl.BlockSpec((4, BN, F), lambda i: (0, i, 0)),
            pl.BlockSpec((H, H), lambda i: (0, 0)),
            pl.BlockSpec((H, H), lambda i: (0, 0)),
            pl.BlockSpec((1, H), lambda i: (0, 0)),
            pl.BlockSpec((1, 1, BN), lambda i: (i, 0, 0)),
        ],
        out_specs=[
            pl.BlockSpec((G, H), lambda i: (0, 0)),
            pl.BlockSpec((G, H), lambda i: (0, 0)),
        ],
        out_shape=[
            jax.ShapeDtypeStruct((G, H), jnp.float32),
            jax.ShapeDtypeStruct((G, H), jnp.float32),
        ],
    )(acc2, h1b, W_rel, W_root, b_rel.reshape(1, H), batch2d)


def _tc_head(pooled, counts, W1, b1, W2, b2, W3, b3):
    def body(p_ref, c_ref, w1_ref, b1_ref, w2_ref, b2_ref, w3_ref, b3_ref,
             out_ref):
        pm = p_ref[...] / jnp.maximum(c_ref[...], 1.0)
        z = lax.dot_general(pm, w1_ref[...], (((1,), (1,)), ((), ())),
                            preferred_element_type=jnp.float32, precision=lax.Precision.HIGHEST) + b1_ref[...]
        z = jnp.maximum(z, 0.0)
        z = lax.dot_general(z, w2_ref[...], (((1,), (1,)), ((), ())),
                            preferred_element_type=jnp.float32, precision=lax.Precision.HIGHEST) + b2_ref[...]
        z = jnp.maximum(z, 0.0)
        out_ref[...] = (jnp.sum(z * w3_ref[...], axis=1, keepdims=True)
                        + b3_ref[...])

    W2p = jnp.zeros((128, G), jnp.float32).at[:16].set(W2)
    b2p = jnp.zeros((1, 128), jnp.float32).at[:, :16].set(b2.reshape(1, 16))
    W3p = jnp.zeros((1, 128), jnp.float32).at[:, :16].set(W3)
    return pl.pallas_call(
        body,
        out_shape=jax.ShapeDtypeStruct((G, 1), jnp.float32),
    )(pooled, counts, W1, b1.reshape(1, G), W2p, b2p, W3p, b3.reshape(1, 1))


def kernel(x, edge_index, edge_attr, batch,
           W_rel1, b_rel1, W_root1,
           W_rel2, b_rel2, W_root2,
           W1, b1, W2, b2, W3, b3):
    pad = E_PAD - E
    src_p = jnp.concatenate([edge_index[0], jnp.zeros((pad,), jnp.int32)])
    dst_p = jnp.concatenate([edge_index[1], jnp.zeros((pad,), jnp.int32)])
    w_p = jnp.concatenate([edge_attr, jnp.zeros((pad,), jnp.float32)])
    idx_pack = jnp.stack([src_p.reshape(NW * NB, B),
                          dst_p.reshape(NW * NB, B)], axis=1)
    w_pack = w_p.reshape(NW * NB, B)

    acc1 = _sc_agg(x, idx_pack, w_pack, C=1).reshape(2, N, F)
    h1b = _tc_layer1(acc1, x, W_rel1, W_root1, b_rel1)
    acc2 = _sc_agg(h1b.reshape(4 * N, F), idx_pack, w_pack,
                   C=4).reshape(4, 2, N, F)
    pooled, counts = _tc_layer2_pool(acc2, h1b, W_rel2, W_root2, b_rel2,
                                     batch.reshape(N // BN, 1, BN))
    return _tc_head(pooled, counts, W1, b1, W2, b2, W3, b3)
